# baseline (device time: 60118 ns/iter reference)
import jax
import jax.numpy as jnp
from jax import lax
from jax.experimental import pallas as pl
from jax.experimental.pallas import tpu as pltpu

N_DEV = 4
N_EXP = 16
EXP_PER_DEV = 4
CAPACITY = 204
CAP_G = 384
NSLOT = N_DEV * CAP_G
META_L = 128


def kernel(x, router_W, route_idx, expert_W):
    del router_W
    n_tok, d_model = x.shape
    _, _, d_ff = expert_W.shape

    xb = x.astype(jnp.bfloat16)
    wb = expert_W.astype(jnp.bfloat16)
    route_col = route_idx
    route_row = route_idx.reshape(1, n_tok)

    D_BLK = d_model + META_L

    def body(
        xb_ref, wb_ref, rc_ref, rr_ref, out_ref,
        xc_s, xin_s, yb_s, yin_s, cnt_s,
        cnt_send, cnt_recv, xa_send, xa_recv, ya_send, ya_recv,
        counts_all,
    ):
        my = lax.axis_index("i")

        barrier = pltpu.get_barrier_semaphore()
        for o in range(1, N_DEV):
            pl.semaphore_signal(
                barrier, inc=1,
                device_id=(jnp.mod(my + o, N_DEV),),
                device_id_type=pl.DeviceIdType.MESH,
            )
        pl.semaphore_wait(barrier, N_DEV - 1)

        rc = rc_ref[...]
        rr = rr_ref[...]

        oh128 = (
            rc == lax.broadcasted_iota(jnp.int32, (n_tok, 128), 1)
        ).astype(jnp.float32)
        cnt_s[...] = oh128.sum(axis=0, keepdims=True)

        cnt_sends = []
        for o in range(1, N_DEV):
            tgt = jnp.mod(my + o, N_DEV)
            s = pltpu.make_async_remote_copy(
                src_ref=cnt_s,
                dst_ref=counts_all.at[pl.ds(my, 1)],
                send_sem=cnt_send.at[o],
                recv_sem=cnt_recv.at[my],
                device_id=(tgt,),
                device_id_type=pl.DeviceIdType.MESH,
            )
            s.start()
            cnt_sends.append(s)

        lower = (
            lax.broadcasted_iota(jnp.int32, (n_tok, n_tok), 0)
            >= lax.broadcasted_iota(jnp.int32, (n_tok, n_tok), 1)
        ).astype(jnp.bfloat16)

        oh16 = (
            rc == lax.broadcasted_iota(jnp.int32, (n_tok, N_EXP), 1)
        )
        oh16b = oh16.astype(jnp.bfloat16)
        oh16f = oh16.astype(jnp.float32)
        csum16 = jnp.dot(lower, oh16b, preferred_element_type=jnp.float32)
        rank = ((csum16 - oh16f) * oh16f).sum(axis=1, keepdims=True)

        gc = rc // EXP_PER_DEV
        ohg_c = (
            gc == lax.broadcasted_iota(jnp.int32, (n_tok, N_DEV), 1)
        )
        ohg_cb = ohg_c.astype(jnp.bfloat16)
        ohg_cf = ohg_c.astype(jnp.float32)
        csg_c = jnp.dot(lower, ohg_cb, preferred_element_type=jnp.float32)
        rkg_c = ((csg_c - ohg_cf) * ohg_cf).sum(axis=1, keepdims=True)
        slot_col = jnp.where(
            rkg_c < CAP_G,
            gc.astype(jnp.float32) * CAP_G + rkg_c,
            float(NSLOT),
        )

        upper = (
            lax.broadcasted_iota(jnp.int32, (n_tok, n_tok), 0)
            <= lax.broadcasted_iota(jnp.int32, (n_tok, n_tok), 1)
        ).astype(jnp.bfloat16)
        gr = rr // EXP_PER_DEV
        ohg_r = (
            gr == lax.broadcasted_iota(jnp.int32, (N_DEV, n_tok), 0)
        )
        ohg_rb = ohg_r.astype(jnp.bfloat16)
        ohg_rf = ohg_r.astype(jnp.float32)
        csg_r = jnp.dot(ohg_rb, upper, preferred_element_type=jnp.float32)
        rkg_r = ((csg_r - ohg_rf) * ohg_rf).sum(axis=0, keepdims=True)
        slot_row = jnp.where(
            rkg_r < CAP_G,
            gr.astype(jnp.float32) * CAP_G + rkg_r,
            float(NSLOT),
        )

        rhi = jnp.floor(rank / 256.0)
        rlo = rank - 256.0 * rhi
        lane = lax.broadcasted_iota(jnp.int32, (1, META_L), 1)
        meta = (
            rc.astype(jnp.float32) * (lane == 0)
            + rhi * (lane == 1)
            + rlo * (lane == 2)
        ).astype(jnp.bfloat16)

        pmat = (
            lax.broadcasted_iota(jnp.int32, (NSLOT, n_tok), 0).astype(
                jnp.float32
            )
            == slot_row
        ).astype(jnp.bfloat16)
        xmeta = jnp.concatenate([xb_ref[...], meta], axis=1)
        xc_s[...] = (
            jnp.dot(pmat, xmeta, preferred_element_type=jnp.float32)
            .astype(jnp.bfloat16)
            .reshape(N_DEV, CAP_G, D_BLK)
        )

        xin_s[pl.ds(my, 1)] = xc_s[pl.ds(my, 1)]
        x_sends = []
        for o in range(1, N_DEV):
            tgt = jnp.mod(my + o, N_DEV)
            sx = pltpu.make_async_remote_copy(
                src_ref=xc_s.at[pl.ds(tgt, 1)],
                dst_ref=xin_s.at[pl.ds(my, 1)],
                send_sem=xa_send.at[o],
                recv_sem=xa_recv.at[my],
                device_id=(tgt,),
                device_id_type=pl.DeviceIdType.MESH,
            )
            sx.start()
            x_sends.append(sx)

        counts_all[pl.ds(my, 1)] = cnt_s[...]
        for o in range(1, N_DEV):
            src_dev = jnp.mod(my + o, N_DEV)
            r = pltpu.make_async_remote_copy(
                src_ref=cnt_s,
                dst_ref=counts_all.at[pl.ds(src_dev, 1)],
                send_sem=cnt_send.at[0],
                recv_sem=cnt_recv.at[src_dev],
                device_id=(src_dev,),
                device_id_type=pl.DeviceIdType.MESH,
            )
            r.wait_recv()

        ca = counts_all[...]
        dev_iota = lax.broadcasted_iota(jnp.int32, (N_DEV, 128), 0)
        lane16 = lax.broadcasted_iota(jnp.int32, (1, N_EXP), 1).astype(
            jnp.float32
        )

        wflat = wb_ref[...].reshape(EXP_PER_DEV * d_model, d_ff)

        def compute_block(s, dst_ref):
            blk = xin_s[pl.ds(s, 1)].reshape(CAP_G, D_BLK)
            xv = blk[:, :d_model]
            ev = blk[:, d_model:d_model + 1].astype(jnp.float32)
            rk = (
                blk[:, d_model + 1:d_model + 2].astype(jnp.float32) * 256.0
                + blk[:, d_model + 2:d_model + 3].astype(jnp.float32)
            )
            offs_vec = jnp.where(dev_iota < s, ca, 0.0).sum(
                axis=0, keepdims=True
            )[:, :N_EXP]
            ohs = (ev == lane16).astype(jnp.float32)
            offs_row = (ohs * offs_vec).sum(axis=1, keepdims=True)
            keep = (rk + offs_row) < float(CAPACITY)
            xms = []
            for k in range(EXP_PER_DEV):
                e_f = (my * EXP_PER_DEV + k).astype(jnp.float32)
                m = (ev == e_f) & keep
                xms.append(xv * m.astype(jnp.bfloat16))
            x4 = jnp.concatenate(xms, axis=1)
            acc = jnp.dot(x4, wflat, preferred_element_type=jnp.float32)
            dst_ref[pl.ds(s, 1)] = acc.astype(jnp.bfloat16)[None]

        compute_block(my, yin_s)

        y_sends = []
        for o in (1, 3, 2):
            s = jnp.mod(my + o, N_DEV)
            rx = pltpu.make_async_remote_copy(
                src_ref=xc_s.at[pl.ds(s, 1)],
                dst_ref=xin_s.at[pl.ds(s, 1)],
                send_sem=xa_send.at[0],
                recv_sem=xa_recv.at[s],
                device_id=(s,),
                device_id_type=pl.DeviceIdType.MESH,
            )
            rx.wait_recv()
            compute_block(s, yb_s)
            sy = pltpu.make_async_remote_copy(
                src_ref=yb_s.at[pl.ds(s, 1)],
                dst_ref=yin_s.at[pl.ds(my, 1)],
                send_sem=ya_send.at[o],
                recv_sem=ya_recv.at[my],
                device_id=(s,),
                device_id_type=pl.DeviceIdType.MESH,
            )
            sy.start()
            y_sends.append(sy)

        diag = jnp.mod(my + 2, N_DEV)
        ptmat = (
            slot_col
            == lax.broadcasted_iota(jnp.int32, (n_tok, NSLOT), 1).astype(
                jnp.float32
            )
        ).astype(jnp.bfloat16)
        slot_g = jnp.floor(slot_col / float(CAP_G))
        pt_early = ptmat * (slot_g != diag.astype(jnp.float32)).astype(
            jnp.bfloat16
        )
        pt_diag = (
            slot_col - diag.astype(jnp.float32) * CAP_G
            == lax.broadcasted_iota(jnp.int32, (n_tok, CAP_G), 1).astype(
                jnp.float32
            )
        ).astype(jnp.bfloat16)

        def wait_y(g):
            ry = pltpu.make_async_remote_copy(
                src_ref=yb_s.at[pl.ds(g, 1)],
                dst_ref=yin_s.at[pl.ds(g, 1)],
                send_sem=ya_send.at[0],
                recv_sem=ya_recv.at[g],
                device_id=(g,),
                device_id_type=pl.DeviceIdType.MESH,
            )
            ry.wait_recv()

        wait_y(jnp.mod(my + 3, N_DEV))
        wait_y(jnp.mod(my + 1, N_DEV))

        yf = yin_s[...].reshape(NSLOT, d_ff)
        partial = jnp.dot(pt_early, yf, preferred_element_type=jnp.float32)

        wait_y(diag)
        y_diag = yin_s[pl.ds(diag, 1)].reshape(CAP_G, d_ff)
        out_ref[...] = partial + jnp.dot(
            pt_diag, y_diag, preferred_element_type=jnp.float32
        )

        for s in cnt_sends + x_sends + y_sends:
            s.wait_send()

    return pl.pallas_call(
        body,
        out_shape=jax.ShapeDtypeStruct((n_tok, d_ff), jnp.float32),
        in_specs=[pl.BlockSpec(memory_space=pltpu.VMEM)] * 4,
        out_specs=pl.BlockSpec(memory_space=pltpu.VMEM),
        scratch_shapes=[
            pltpu.VMEM((N_DEV, CAP_G, D_BLK), jnp.bfloat16),
            pltpu.VMEM((N_DEV, CAP_G, D_BLK), jnp.bfloat16),
            pltpu.VMEM((N_DEV, CAP_G, d_ff), jnp.bfloat16),
            pltpu.VMEM((N_DEV, CAP_G, d_ff), jnp.bfloat16),
            pltpu.VMEM((1, 128), jnp.float32),
            pltpu.SemaphoreType.DMA((N_DEV,)),
            pltpu.SemaphoreType.DMA((N_DEV,)),
            pltpu.SemaphoreType.DMA((N_DEV,)),
            pltpu.SemaphoreType.DMA((N_DEV,)),
            pltpu.SemaphoreType.DMA((N_DEV,)),
            pltpu.SemaphoreType.DMA((N_DEV,)),
            pltpu.VMEM((N_DEV, 128), jnp.float32),
        ],
        compiler_params=pltpu.CompilerParams(collective_id=0),
    )(xb, wb, route_col, route_row)


# device time: 58977 ns/iter; 1.0193x vs baseline; 1.0193x over previous
import jax
import jax.numpy as jnp
from jax import lax
from jax.experimental import pallas as pl
from jax.experimental.pallas import tpu as pltpu

N_DEV = 4
N_EXP = 16
EXP_PER_DEV = 4
CAPACITY = 204
CAP_G = 384
NSLOT = N_DEV * CAP_G
META_L = 128


def kernel(x, router_W, route_idx, expert_W):
    del router_W
    n_tok, d_model = x.shape
    _, _, d_ff = expert_W.shape

    xb = x.astype(jnp.bfloat16)
    wb = expert_W.astype(jnp.bfloat16)
    route_col = route_idx
    route_row = route_idx.reshape(1, n_tok)

    D_BLK = d_model + META_L

    def body(
        xb_ref, wb_ref, rc_ref, rr_ref, out_ref,
        xc_s, xin_s, yb_s, yin_s, cnt_s,
        cnt_send, cnt_recv, xa_send, xa_recv, ya_send, ya_recv,
        counts_all,
    ):
        my = lax.axis_index("i")

        barrier = pltpu.get_barrier_semaphore()
        for o in range(1, N_DEV):
            pl.semaphore_signal(
                barrier, inc=1,
                device_id=(jnp.mod(my + o, N_DEV),),
                device_id_type=pl.DeviceIdType.MESH,
            )
        pl.semaphore_wait(barrier, N_DEV - 1)

        rc = rc_ref[...]
        rr = rr_ref[...]

        oh128 = (
            rc == lax.broadcasted_iota(jnp.int32, (n_tok, 128), 1)
        ).astype(jnp.float32)
        cnt_s[...] = oh128.sum(axis=0, keepdims=True)

        cnt_sends = []
        for o in range(1, N_DEV):
            tgt = jnp.mod(my + o, N_DEV)
            s = pltpu.make_async_remote_copy(
                src_ref=cnt_s,
                dst_ref=counts_all.at[pl.ds(my, 1)],
                send_sem=cnt_send.at[o],
                recv_sem=cnt_recv.at[my],
                device_id=(tgt,),
                device_id_type=pl.DeviceIdType.MESH,
            )
            s.start()
            cnt_sends.append(s)

        lower = (
            lax.broadcasted_iota(jnp.int32, (n_tok, n_tok), 0)
            >= lax.broadcasted_iota(jnp.int32, (n_tok, n_tok), 1)
        ).astype(jnp.bfloat16)

        oh16 = (
            rc == lax.broadcasted_iota(jnp.int32, (n_tok, N_EXP), 1)
        )
        oh16b = oh16.astype(jnp.bfloat16)
        oh16f = oh16.astype(jnp.float32)
        csum16 = jnp.dot(lower, oh16b, preferred_element_type=jnp.float32)
        rank = ((csum16 - oh16f) * oh16f).sum(axis=1, keepdims=True)

        gc = rc // EXP_PER_DEV
        ohg_c = (
            gc == lax.broadcasted_iota(jnp.int32, (n_tok, N_DEV), 1)
        )
        ohg_cb = ohg_c.astype(jnp.bfloat16)
        ohg_cf = ohg_c.astype(jnp.float32)
        csg_c = jnp.dot(lower, ohg_cb, preferred_element_type=jnp.float32)
        rkg_c = ((csg_c - ohg_cf) * ohg_cf).sum(axis=1, keepdims=True)
        slot_col = jnp.where(
            rkg_c < CAP_G,
            gc.astype(jnp.float32) * CAP_G + rkg_c,
            float(NSLOT),
        )

        upper = (
            lax.broadcasted_iota(jnp.int32, (n_tok, n_tok), 0)
            <= lax.broadcasted_iota(jnp.int32, (n_tok, n_tok), 1)
        ).astype(jnp.bfloat16)
        gr = rr // EXP_PER_DEV
        ohg_r = (
            gr == lax.broadcasted_iota(jnp.int32, (N_DEV, n_tok), 0)
        )
        ohg_rb = ohg_r.astype(jnp.bfloat16)
        ohg_rf = ohg_r.astype(jnp.float32)
        csg_r = jnp.dot(ohg_rb, upper, preferred_element_type=jnp.float32)
        rkg_r = ((csg_r - ohg_rf) * ohg_rf).sum(axis=0, keepdims=True)
        slot_row = jnp.where(
            rkg_r < CAP_G,
            gr.astype(jnp.float32) * CAP_G + rkg_r,
            float(NSLOT),
        )

        rhi = jnp.floor(rank / 256.0)
        rlo = rank - 256.0 * rhi
        lane = lax.broadcasted_iota(jnp.int32, (1, META_L), 1)
        meta = (
            rc.astype(jnp.float32) * (lane == 0)
            + rhi * (lane == 1)
            + rlo * (lane == 2)
        ).astype(jnp.bfloat16)

        pmat = (
            lax.broadcasted_iota(jnp.int32, (NSLOT, n_tok), 0).astype(
                jnp.float32
            )
            == slot_row
        ).astype(jnp.bfloat16)
        xmeta = jnp.concatenate([xb_ref[...], meta], axis=1)
        xc_s[...] = (
            jnp.dot(pmat, xmeta, preferred_element_type=jnp.float32)
            .astype(jnp.bfloat16)
            .reshape(N_DEV, CAP_G, D_BLK)
        )

        xin_s[pl.ds(my, 1)] = xc_s[pl.ds(my, 1)]
        x_sends = []
        for o in range(1, N_DEV):
            tgt = jnp.mod(my + o, N_DEV)
            sx = pltpu.make_async_remote_copy(
                src_ref=xc_s.at[pl.ds(tgt, 1)],
                dst_ref=xin_s.at[pl.ds(my, 1)],
                send_sem=xa_send.at[o],
                recv_sem=xa_recv.at[my],
                device_id=(tgt,),
                device_id_type=pl.DeviceIdType.MESH,
            )
            sx.start()
            x_sends.append(sx)

        counts_all[pl.ds(my, 1)] = cnt_s[...]
        for o in range(1, N_DEV):
            src_dev = jnp.mod(my + o, N_DEV)
            r = pltpu.make_async_remote_copy(
                src_ref=cnt_s,
                dst_ref=counts_all.at[pl.ds(src_dev, 1)],
                send_sem=cnt_send.at[0],
                recv_sem=cnt_recv.at[src_dev],
                device_id=(src_dev,),
                device_id_type=pl.DeviceIdType.MESH,
            )
            r.wait_recv()

        ca = counts_all[...]
        dev_iota = lax.broadcasted_iota(jnp.int32, (N_DEV, 128), 0)
        lane16 = lax.broadcasted_iota(jnp.int32, (1, N_EXP), 1).astype(
            jnp.float32
        )

        wflat = wb_ref[...].reshape(EXP_PER_DEV * d_model, d_ff)

        def compute_block(s, dst_ref):
            blk = xin_s[pl.ds(s, 1)].reshape(CAP_G, D_BLK)
            xv = blk[:, :d_model]
            ev = blk[:, d_model:d_model + 1].astype(jnp.float32)
            rk = (
                blk[:, d_model + 1:d_model + 2].astype(jnp.float32) * 256.0
                + blk[:, d_model + 2:d_model + 3].astype(jnp.float32)
            )
            offs_vec = jnp.where(dev_iota < s, ca, 0.0).sum(
                axis=0, keepdims=True
            )[:, :N_EXP]
            ohs = (ev == lane16).astype(jnp.float32)
            offs_row = (ohs * offs_vec).sum(axis=1, keepdims=True)
            keep = (rk + offs_row) < float(CAPACITY)
            xms = []
            for k in range(EXP_PER_DEV):
                e_f = (my * EXP_PER_DEV + k).astype(jnp.float32)
                m = (ev == e_f) & keep
                xms.append(xv * m.astype(jnp.bfloat16))
            x4 = jnp.concatenate(xms, axis=1)
            acc = jnp.dot(x4, wflat, preferred_element_type=jnp.float32)
            dst_ref[pl.ds(s, 1)] = acc.astype(jnp.bfloat16)[None]

        compute_block(my, yin_s)

        y_sends = []
        for o in (1, 3, 2):
            s = jnp.mod(my + o, N_DEV)
            rx = pltpu.make_async_remote_copy(
                src_ref=xc_s.at[pl.ds(s, 1)],
                dst_ref=xin_s.at[pl.ds(s, 1)],
                send_sem=xa_send.at[0],
                recv_sem=xa_recv.at[s],
                device_id=(s,),
                device_id_type=pl.DeviceIdType.MESH,
            )
            rx.wait_recv()
            compute_block(s, yb_s)
            sy = pltpu.make_async_remote_copy(
                src_ref=yb_s.at[pl.ds(s, 1)],
                dst_ref=yin_s.at[pl.ds(my, 1)],
                send_sem=ya_send.at[o],
                recv_sem=ya_recv.at[my],
                device_id=(s,),
                device_id_type=pl.DeviceIdType.MESH,
            )
            sy.start()
            y_sends.append(sy)

        ptmat = (
            slot_col
            == lax.broadcasted_iota(jnp.int32, (n_tok, NSLOT), 1).astype(
                jnp.float32
            )
        ).astype(jnp.bfloat16)

        for o in (1, 3, 2):
            g = jnp.mod(my + o, N_DEV)
            ry = pltpu.make_async_remote_copy(
                src_ref=yb_s.at[pl.ds(g, 1)],
                dst_ref=yin_s.at[pl.ds(g, 1)],
                send_sem=ya_send.at[0],
                recv_sem=ya_recv.at[g],
                device_id=(g,),
                device_id_type=pl.DeviceIdType.MESH,
            )
            ry.wait_recv()

        yf = yin_s[...].reshape(NSLOT, d_ff)
        out_ref[...] = jnp.dot(
            ptmat, yf, preferred_element_type=jnp.float32
        )

        for s in cnt_sends + x_sends + y_sends:
            s.wait_send()

    return pl.pallas_call(
        body,
        out_shape=jax.ShapeDtypeStruct((n_tok, d_ff), jnp.float32),
        in_specs=[pl.BlockSpec(memory_space=pltpu.VMEM)] * 4,
        out_specs=pl.BlockSpec(memory_space=pltpu.VMEM),
        scratch_shapes=[
            pltpu.VMEM((N_DEV, CAP_G, D_BLK), jnp.bfloat16),
            pltpu.VMEM((N_DEV, CAP_G, D_BLK), jnp.bfloat16),
            pltpu.VMEM((N_DEV, CAP_G, d_ff), jnp.bfloat16),
            pltpu.VMEM((N_DEV, CAP_G, d_ff), jnp.bfloat16),
            pltpu.VMEM((1, 128), jnp.float32),
            pltpu.SemaphoreType.DMA((N_DEV,)),
            pltpu.SemaphoreType.DMA((N_DEV,)),
            pltpu.SemaphoreType.DMA((N_DEV,)),
            pltpu.SemaphoreType.DMA((N_DEV,)),
            pltpu.SemaphoreType.DMA((N_DEV,)),
            pltpu.SemaphoreType.DMA((N_DEV,)),
            pltpu.VMEM((N_DEV, 128), jnp.float32),
        ],
        compiler_params=pltpu.CompilerParams(collective_id=0),
    )(xb, wb, route_col, route_row)
